# Initial kernel scaffold; baseline (speedup 1.0000x reference)
import jax
import jax.numpy as jnp
from jax.experimental import pallas as pl


def kernel(vertices, control_def, neighbours, neighbour_dists):
    raise NotImplementedError("write your pallas kernel here")


# speedup vs baseline: 21.7004x; 21.7004x over previous
import functools

import jax
import jax.numpy as jnp
from jax import lax
from jax.experimental import pallas as pl
from jax.experimental.pallas import tpu as pltpu
from jax.experimental.pallas import tpu_sc as plsc

_N = 100000
_C = 3125
_K = 25
_CPAD = 3128
_NC = 2
_NS = 16
_NW = _NC * _NS
_L = 16

_VPW = 3136
_SUB = 784
_NSUB = _VPW // _SUB
_NB = _SUB // _L


def _body(vert_hbm, ctrl_hbm, nbr_hbm, dist_hbm, out_hbm,
          ctrl_v, nbr_v, dist_v, vert_v, out_v):
    wid = lax.axis_index("s") * _NC + lax.axis_index("c")
    start = jnp.minimum(wid * _VPW, _N - _VPW)

    pltpu.sync_copy(ctrl_hbm, ctrl_v)

    iota = lax.iota(jnp.int32, _L)
    iota_k = iota * _K
    iota_3 = iota * 3
    row0 = jnp.zeros((_L,), jnp.int32)
    row1 = row0 + 1
    row2 = row0 + 2

    for sub in range(_NSUB):
        s0 = start + sub * _SUB
        pltpu.sync_copy(nbr_hbm.at[pl.ds(pl.multiple_of(s0 * _K, 8), _SUB * _K)],
                        nbr_v)
        pltpu.sync_copy(dist_hbm.at[pl.ds(pl.multiple_of(s0 * _K, 8), _SUB * _K)],
                        dist_v)
        pltpu.sync_copy(vert_hbm.at[pl.ds(pl.multiple_of(s0 * 3, 8), _SUB * 3)],
                        vert_v)

        def block(b, carry):
            kbase = b * (_L * _K) + iota_k
            ax = jnp.zeros((_L,), jnp.float32)
            ay = jnp.zeros((_L,), jnp.float32)
            az = jnp.zeros((_L,), jnp.float32)
            ws = jnp.zeros((_L,), jnp.float32)
            for j in range(_K):
                col = kbase + j
                nb = plsc.load_gather(nbr_v, [col])
                dj = plsc.load_gather(dist_v, [col])
                w = jnp.exp(dj * (-4.5))
                ws = ws + w
                ax = ax + w * plsc.load_gather(ctrl_v, [row0, nb])
                ay = ay + w * plsc.load_gather(ctrl_v, [row1, nb])
                az = az + w * plsc.load_gather(ctrl_v, [row2, nb])
            inv = 1.0 / jnp.maximum(ws, 0.01)
            vbase = b * (_L * 3) + iota_3
            vx = plsc.load_gather(vert_v, [vbase])
            vy = plsc.load_gather(vert_v, [vbase + 1])
            vz = plsc.load_gather(vert_v, [vbase + 2])
            plsc.store_scatter(out_v, [vbase], vx + ax * inv)
            plsc.store_scatter(out_v, [vbase + 1], vy + ay * inv)
            plsc.store_scatter(out_v, [vbase + 2], vz + az * inv)
            return carry

        lax.fori_loop(0, _NB, block, 0)
        pltpu.sync_copy(out_v,
                        out_hbm.at[pl.ds(pl.multiple_of(s0 * 3, 8), _SUB * 3)])


_mesh = plsc.VectorSubcoreMesh(core_axis_name="c", subcore_axis_name="s")

_sc_call = functools.partial(
    pl.kernel,
    mesh=_mesh,
    compiler_params=pltpu.CompilerParams(needs_layout_passes=False),
    out_type=jax.ShapeDtypeStruct((_N * 3,), jnp.float32),
    scratch_types=[
        pltpu.VMEM((3, _CPAD), jnp.float32),
        pltpu.VMEM((_SUB * _K,), jnp.int32),
        pltpu.VMEM((_SUB * _K,), jnp.float32),
        pltpu.VMEM((_SUB * 3,), jnp.float32),
        pltpu.VMEM((_SUB * 3,), jnp.float32),
    ],
)(_body)


def kernel(vertices, control_def, neighbours, neighbour_dists):
    ctrl = jnp.pad(control_def.T, ((0, 0), (0, _CPAD - _C)))
    nbr = neighbours.astype(jnp.int32).reshape(-1)
    dist = neighbour_dists.reshape(-1)
    vert = vertices.reshape(-1)
    out = _sc_call(vert, ctrl, nbr, dist)
    return out.reshape(_N, 3)
